# Initial kernel scaffold; baseline (speedup 1.0000x reference)
import jax
import jax.numpy as jnp
from jax.experimental import pallas as pl


def kernel(inputs, table):
    raise NotImplementedError("write your pallas kernel here")


# speedup vs baseline: 1.5759x; 1.5759x over previous
import functools

import jax
import jax.numpy as jnp
from jax import lax
from jax.experimental import pallas as pl
from jax.experimental.pallas import tpu as pltpu
from jax.experimental.pallas import tpu_sc as plsc

EMBED_DIM = 32
CHUNK = 1024


@functools.lru_cache(maxsize=None)
def _make_gather(n_rows: int, vocab: int, dim: int):
    info = plsc.get_sparse_core_info()
    num_cores, num_subcores = info.num_cores, info.num_subcores
    num_workers = num_cores * num_subcores
    rows_per_worker = n_rows // num_workers
    assert rows_per_worker * num_workers == n_rows
    chunk = min(CHUNK, rows_per_worker)
    n_chunks = rows_per_worker // chunk
    assert n_chunks * chunk == rows_per_worker

    mesh = plsc.VectorSubcoreMesh(core_axis_name="c", subcore_axis_name="s")

    @functools.partial(
        pl.kernel,
        out_type=jax.ShapeDtypeStruct((n_rows, dim), jnp.float32),
        mesh=mesh,
        scratch_types=[
            pltpu.VMEM((rows_per_worker,), jnp.int32),
            pltpu.VMEM((2, chunk, dim), jnp.float32),
            pltpu.SemaphoreType.DMA,
            pltpu.SemaphoreType.DMA,
        ],
        compiler_params=pltpu.CompilerParams(use_tc_tiling_on_sc=False),
    )
    def gather_kernel(idx_hbm, table_hbm, out_hbm, idx_v, rows_v, sem0, sem1):
        wid = lax.axis_index("s") * num_cores + lax.axis_index("c")
        base = wid * rows_per_worker
        pltpu.sync_copy(idx_hbm.at[pl.ds(base, rows_per_worker)], idx_v)

        sems = (sem0, sem1)
        copies = [None, None]
        copies[0] = pltpu.async_copy(
            table_hbm.at[idx_v.at[pl.ds(0, chunk)]], rows_v.at[0], sems[0]
        )
        for j in range(n_chunks):
            buf = j % 2
            if j + 1 < n_chunks:
                nbuf = (j + 1) % 2
                copies[nbuf] = pltpu.async_copy(
                    table_hbm.at[idx_v.at[pl.ds((j + 1) * chunk, chunk)]],
                    rows_v.at[nbuf],
                    sems[nbuf],
                )
            copies[buf].wait()
            pltpu.sync_copy(
                rows_v.at[buf], out_hbm.at[pl.ds(base + j * chunk, chunk)]
            )

    return gather_kernel


def kernel(inputs, table):
    batch, fields = inputs.shape
    vocab, dim = table.shape
    idx_flat = inputs.reshape(-1).astype(jnp.int32)
    gather = _make_gather(idx_flat.shape[0], vocab, dim)
    out = gather(idx_flat, table)
    return out.reshape(batch, fields, dim)
